# baseline (speedup 1.0000x reference)
import jax
import jax.numpy as jnp
from jax.experimental import pallas as pl
from jax.experimental.pallas import tpu as pltpu

_EPS = 1e-5
_SLOPE = 0.1
_MM = jnp.bfloat16


def _tap_matmuls(cat_ref, w_ref, H, W, C):
    acc = None
    for dy in range(3):
        slab = cat_ref[dy:dy + H].reshape(H * W, 3 * C)
        d = jnp.dot(slab, w_ref[dy], preferred_element_type=jnp.float32)
        acc = d if acc is None else acc + d
    return acc


def _scale_shift(st_ref, g_ref, b_ref, r_total):
    mean = st_ref[0:1] / r_total
    var = st_ref[1:2] / r_total - mean * mean
    scale = g_ref[...] * jax.lax.rsqrt(var + _EPS)
    shift = b_ref[...] - mean * scale
    return scale, shift


def _fused_kernel(xp_ref, w1_ref, w2_ref, g1_ref, b1_ref, g2_ref, b2_ref,
                  o_ref,
                  y1_ref, y2_ref, st1_ref, st2_ref, xc_ref, hc_ref):
    p = pl.program_id(0)
    i = pl.program_id(1)
    N = pl.num_programs(1)
    H = y1_ref.shape[1]
    W = y1_ref.shape[2]
    C1 = xp_ref.shape[3]
    Cm = y1_ref.shape[3]
    C2 = y2_ref.shape[2]
    R = float(N * H * W)

    @pl.when(p == 0)
    def _phase0():
        @pl.when(i == 0)
        def _():
            st1_ref[...] = jnp.zeros_like(st1_ref)
            st2_ref[...] = jnp.zeros_like(st2_ref)
            hc_ref[0] = jnp.zeros((W, 3 * Cm), _MM)
            hc_ref[H + 1] = jnp.zeros((W, 3 * Cm), _MM)
        for dx in range(3):
            xc_ref[:, :, dx * C1:(dx + 1) * C1] = xp_ref[0, :, dx:dx + W, :]
        acc = _tap_matmuls(xc_ref, w1_ref, H, W, C1)
        y1_ref[pl.ds(i, 1)] = acc.reshape(1, H, W, Cm).astype(_MM)
        st1_ref[0:1] = st1_ref[0:1] + jnp.sum(acc, axis=0, keepdims=True)
        st1_ref[1:2] = st1_ref[1:2] + jnp.sum(acc * acc, axis=0,
                                              keepdims=True)

    @pl.when(p == 1)
    def _phase1():
        scale, shift = _scale_shift(st1_ref, g1_ref, b1_ref, R)
        h = (y1_ref[pl.ds(i, 1)][0].astype(jnp.float32)
             * scale.reshape(1, 1, Cm) + shift.reshape(1, 1, Cm))
        h = jnp.where(h >= 0.0, h, _SLOPE * h)
        hp = jnp.pad(h.astype(_MM), ((0, 0), (1, 1), (0, 0)))
        for dx in range(3):
            hc_ref[1:H + 1, :, dx * Cm:(dx + 1) * Cm] = hp[:, dx:dx + W, :]
        acc = _tap_matmuls(hc_ref, w2_ref, H, W, Cm)
        y2_ref[pl.ds(i, 1)] = acc.reshape(1, H * W, C2).astype(_MM)
        st2_ref[0:1] = st2_ref[0:1] + jnp.sum(acc, axis=0, keepdims=True)
        st2_ref[1:2] = st2_ref[1:2] + jnp.sum(acc * acc, axis=0,
                                              keepdims=True)

    @pl.when(p == 2)
    def _phase2():
        scale, shift = _scale_shift(st2_ref, g2_ref, b2_ref, R)
        h = y2_ref[pl.ds(i, 1)][0].astype(jnp.float32) * scale + shift
        o_ref[0] = jnp.where(h >= 0.0, h, _SLOPE * h).astype(_MM)


@jax.jit
def _double_conv(x_nchw, w1, g1, b1, w2, g2, b2):
    N, C1, H, W = x_nchw.shape
    Cm = w1.shape[-1]
    C2 = w2.shape[-1]

    x = jnp.transpose(x_nchw, (0, 2, 3, 1)).astype(_MM)
    xp = jnp.pad(x, ((0, 0), (1, 1), (1, 1), (0, 0)))
    w1c = w1.reshape(3, 3 * C1, Cm).astype(_MM)
    w2c = w2.reshape(3, 3 * Cm, C2).astype(_MM)
    g1r = g1.astype(jnp.float32).reshape(1, Cm)
    b1r = b1.astype(jnp.float32).reshape(1, Cm)
    g2r = g2.astype(jnp.float32).reshape(1, C2)
    b2r = b2.astype(jnp.float32).reshape(1, C2)

    vec = lambda c: pl.BlockSpec((1, c), lambda p, i: (0, 0))

    out = pl.pallas_call(
        _fused_kernel,
        grid=(3, N),
        in_specs=[
            pl.BlockSpec((1, H + 2, W + 2, C1),
                         lambda p, i: (jnp.where(p == 0, i, N - 1), 0, 0, 0)),
            pl.BlockSpec((3, 3 * C1, Cm), lambda p, i: (0, 0, 0)),
            pl.BlockSpec((3, 3 * Cm, C2), lambda p, i: (0, 0, 0)),
            vec(Cm), vec(Cm), vec(C2), vec(C2),
        ],
        out_specs=pl.BlockSpec((1, H * W, C2),
                               lambda p, i: (jnp.where(p == 2, i, 0), 0, 0)),
        out_shape=jax.ShapeDtypeStruct((N, H * W, C2), _MM),
        scratch_shapes=[
            pltpu.VMEM((N, H, W, Cm), _MM),
            pltpu.VMEM((N, H * W, C2), _MM),
            pltpu.VMEM((2, Cm), jnp.float32),
            pltpu.VMEM((2, C2), jnp.float32),
            pltpu.VMEM((H + 2, W, 3 * C1), _MM),
            pltpu.VMEM((H + 2, W, 3 * Cm), _MM),
        ],
        compiler_params=pltpu.CompilerParams(
            dimension_semantics=("arbitrary", "arbitrary"),
            vmem_limit_bytes=60 * 1024 * 1024),
    )(xp, w1c, w2c, g1r, b1r, g2r, b2r)

    return jnp.transpose(out.reshape(N, H, W, C2), (0, 3, 1, 2)).astype(jnp.float32)


def kernel(x_nchw, w1, g1, b1, w2, g2, b2):
    return _double_conv(x_nchw, w1, g1, b1, w2, g2, b2)


# speedup vs baseline: 1.0955x; 1.0955x over previous
import jax
import jax.numpy as jnp
from jax.experimental import pallas as pl
from jax.experimental.pallas import tpu as pltpu

_EPS = 1e-5
_SLOPE = 0.1
_MM = jnp.bfloat16


def _tap_matmuls(cat_ref, w_ref, H, W, C):
    acc = None
    for dy in range(3):
        slab = cat_ref[dy:dy + H].reshape(H * W, 3 * C)
        d = jnp.dot(slab, w_ref[dy], preferred_element_type=jnp.float32)
        acc = d if acc is None else acc + d
    return acc


def _scale_shift(st_ref, g_ref, b_ref, r_total):
    mean = st_ref[0:1] / r_total
    var = st_ref[1:2] / r_total - mean * mean
    scale = g_ref[...] * jax.lax.rsqrt(var + _EPS)
    shift = b_ref[...] - mean * scale
    return scale, shift


def _fused_kernel(xp_ref, w1_ref, w2_ref, g1_ref, b1_ref, g2_ref, b2_ref,
                  o_ref,
                  y1_ref, y2_ref, st1_ref, st2_ref, xc_ref, hc_ref):
    p = pl.program_id(0)
    i = pl.program_id(1)
    N = pl.num_programs(1)
    H = y1_ref.shape[1]
    W = y1_ref.shape[2]
    C1 = xp_ref.shape[3]
    Cm = y1_ref.shape[3]
    C2 = y2_ref.shape[2]
    R = float(N * H * W)

    @pl.when(p == 0)
    def _phase0():
        @pl.when(i == 0)
        def _():
            st1_ref[...] = jnp.zeros_like(st1_ref)
            st2_ref[...] = jnp.zeros_like(st2_ref)
            hc_ref[0] = jnp.zeros((W, 3 * Cm), _MM)
            hc_ref[H + 1] = jnp.zeros((W, 3 * Cm), _MM)
        for dx in range(3):
            xc_ref[:, :, dx * C1:(dx + 1) * C1] = xp_ref[0, :, dx:dx + W, :]
        acc = _tap_matmuls(xc_ref, w1_ref, H, W, C1)
        y1_ref[pl.ds(i, 1)] = acc.reshape(1, H, W, Cm).astype(_MM)
        st1_ref[0:1] = st1_ref[0:1] + jnp.sum(acc, axis=0, keepdims=True)
        st1_ref[1:2] = st1_ref[1:2] + jnp.sum(acc * acc, axis=0,
                                              keepdims=True)

    @pl.when(p == 1)
    def _phase1():
        scale, shift = _scale_shift(st1_ref, g1_ref, b1_ref, R)
        h = (y1_ref[pl.ds(i, 1)][0].astype(jnp.float32)
             * scale.reshape(1, 1, Cm) + shift.reshape(1, 1, Cm))
        h = jnp.where(h >= 0.0, h, _SLOPE * h)
        hp = jnp.pad(h.astype(_MM), ((0, 0), (1, 1), (0, 0)))
        for dx in range(3):
            hc_ref[1:H + 1, :, dx * Cm:(dx + 1) * Cm] = hp[:, dx:dx + W, :]
        acc = _tap_matmuls(hc_ref, w2_ref, H, W, Cm)
        y2_ref[pl.ds(i, 1)] = acc.reshape(1, H * W, C2).astype(_MM)
        st2_ref[0:1] = st2_ref[0:1] + jnp.sum(acc, axis=0, keepdims=True)
        st2_ref[1:2] = st2_ref[1:2] + jnp.sum(acc * acc, axis=0,
                                              keepdims=True)

    @pl.when(p == 2)
    def _phase2():
        scale, shift = _scale_shift(st2_ref, g2_ref, b2_ref, R)
        h = y2_ref[pl.ds(i, 1)][0].astype(jnp.float32) * scale + shift
        o_ref[0] = jnp.where(h >= 0.0, h, _SLOPE * h)


@jax.jit
def _double_conv(x_nchw, w1, g1, b1, w2, g2, b2):
    N, C1, H, W = x_nchw.shape
    Cm = w1.shape[-1]
    C2 = w2.shape[-1]

    x = jnp.transpose(x_nchw, (0, 2, 3, 1)).astype(_MM)
    xp = jnp.pad(x, ((0, 0), (1, 1), (1, 1), (0, 0)))
    w1c = w1.reshape(3, 3 * C1, Cm).astype(_MM)
    w2c = w2.reshape(3, 3 * Cm, C2).astype(_MM)
    g1r = g1.astype(jnp.float32).reshape(1, Cm)
    b1r = b1.astype(jnp.float32).reshape(1, Cm)
    g2r = g2.astype(jnp.float32).reshape(1, C2)
    b2r = b2.astype(jnp.float32).reshape(1, C2)

    vec = lambda c: pl.BlockSpec((1, c), lambda p, i: (0, 0))

    out = pl.pallas_call(
        _fused_kernel,
        grid=(3, N),
        in_specs=[
            pl.BlockSpec((1, H + 2, W + 2, C1),
                         lambda p, i: (jnp.where(p == 0, i, N - 1), 0, 0, 0)),
            pl.BlockSpec((3, 3 * C1, Cm), lambda p, i: (0, 0, 0)),
            pl.BlockSpec((3, 3 * Cm, C2), lambda p, i: (0, 0, 0)),
            vec(Cm), vec(Cm), vec(C2), vec(C2),
        ],
        out_specs=pl.BlockSpec((1, H * W, C2),
                               lambda p, i: (jnp.where(p == 2, i, 0), 0, 0)),
        out_shape=jax.ShapeDtypeStruct((N, H * W, C2), jnp.float32),
        scratch_shapes=[
            pltpu.VMEM((N, H, W, Cm), _MM),
            pltpu.VMEM((N, H * W, C2), _MM),
            pltpu.VMEM((2, Cm), jnp.float32),
            pltpu.VMEM((2, C2), jnp.float32),
            pltpu.VMEM((H + 2, W, 3 * C1), _MM),
            pltpu.VMEM((H + 2, W, 3 * Cm), _MM),
        ],
        compiler_params=pltpu.CompilerParams(
            dimension_semantics=("arbitrary", "arbitrary"),
            vmem_limit_bytes=60 * 1024 * 1024),
    )(xp, w1c, w2c, g1r, b1r, g2r, b2r)

    return jnp.transpose(out.reshape(N, H, W, C2), (0, 3, 1, 2))


def kernel(x_nchw, w1, g1, b1, w2, g2, b2):
    return _double_conv(x_nchw, w1, g1, b1, w2, g2, b2)


# speedup vs baseline: 1.1033x; 1.0071x over previous
import jax
import jax.numpy as jnp
from jax.experimental import pallas as pl
from jax.experimental.pallas import tpu as pltpu

_EPS = 1e-5
_SLOPE = 0.1
_MM = jnp.bfloat16


def _tap_matmuls(cat_ref, w_ref, H, W, C):
    acc = None
    for dy in range(3):
        slab = cat_ref[dy:dy + H].reshape(H * W, 3 * C)
        d = jnp.dot(slab, w_ref[dy], preferred_element_type=jnp.float32)
        acc = d if acc is None else acc + d
    return acc


def _scale_shift(st_ref, g_ref, b_ref, r_total):
    mean = st_ref[0:1] / r_total
    var = st_ref[1:2] / r_total - mean * mean
    scale = g_ref[...] * jax.lax.rsqrt(var + _EPS)
    shift = b_ref[...] - mean * scale
    return scale, shift


def _fused_kernel(xp_ref, w1_ref, w2_ref, g1_ref, b1_ref, g2_ref, b2_ref,
                  o_ref,
                  y1_ref, y2_ref, st1_ref, st2_ref, xc_ref, hc_ref):
    p = pl.program_id(0)
    i = pl.program_id(1)
    N = pl.num_programs(1)
    H = y1_ref.shape[1]
    W = y1_ref.shape[2]
    C1 = xp_ref.shape[3]
    Cm = y1_ref.shape[3]
    C2 = y2_ref.shape[2]
    R = float(N * H * W)

    @pl.when(p == 0)
    def _phase0():
        @pl.when(i == 0)
        def _():
            st1_ref[...] = jnp.zeros_like(st1_ref)
            st2_ref[...] = jnp.zeros_like(st2_ref)
            hc_ref[0] = jnp.zeros((W, 3 * Cm), _MM)
            hc_ref[H + 1] = jnp.zeros((W, 3 * Cm), _MM)
        for dx in range(3):
            xc_ref[:, :, dx * C1:(dx + 1) * C1] = xp_ref[0, :, dx:dx + W, :]
        acc = _tap_matmuls(xc_ref, w1_ref, H, W, C1)
        y1_ref[pl.ds(i, 1)] = acc.reshape(1, H, W, Cm).astype(_MM)
        st1_ref[0:1] = st1_ref[0:1] + jnp.sum(acc, axis=0, keepdims=True)
        st1_ref[1:2] = st1_ref[1:2] + jnp.sum(acc * acc, axis=0,
                                              keepdims=True)

    @pl.when(p == 1)
    def _phase1():
        scale, shift = _scale_shift(st1_ref, g1_ref, b1_ref, R)
        h = (y1_ref[pl.ds(i, 1)][0] * scale.reshape(1, 1, Cm).astype(_MM)
             + shift.reshape(1, 1, Cm).astype(_MM))
        h = jnp.where(h >= 0.0, h, _MM(_SLOPE) * h)
        hp = jnp.pad(h, ((0, 0), (1, 1), (0, 0)))
        for dx in range(3):
            hc_ref[1:H + 1, :, dx * Cm:(dx + 1) * Cm] = hp[:, dx:dx + W, :]
        acc = _tap_matmuls(hc_ref, w2_ref, H, W, Cm)
        y2_ref[pl.ds(i, 1)] = acc.reshape(1, H * W, C2).astype(_MM)
        st2_ref[0:1] = st2_ref[0:1] + jnp.sum(acc, axis=0, keepdims=True)
        st2_ref[1:2] = st2_ref[1:2] + jnp.sum(acc * acc, axis=0,
                                              keepdims=True)

    @pl.when(p == 2)
    def _phase2():
        scale, shift = _scale_shift(st2_ref, g2_ref, b2_ref, R)
        h = y2_ref[pl.ds(i, 1)][0].astype(jnp.float32) * scale + shift
        o_ref[0] = jnp.where(h >= 0.0, h, _SLOPE * h)


@jax.jit
def _double_conv(x_nchw, w1, g1, b1, w2, g2, b2):
    N, C1, H, W = x_nchw.shape
    Cm = w1.shape[-1]
    C2 = w2.shape[-1]

    x = jnp.transpose(x_nchw, (0, 2, 3, 1)).astype(_MM)
    xp = jnp.pad(x, ((0, 0), (1, 1), (1, 1), (0, 0)))
    w1c = w1.reshape(3, 3 * C1, Cm).astype(_MM)
    w2c = w2.reshape(3, 3 * Cm, C2).astype(_MM)
    g1r = g1.astype(jnp.float32).reshape(1, Cm)
    b1r = b1.astype(jnp.float32).reshape(1, Cm)
    g2r = g2.astype(jnp.float32).reshape(1, C2)
    b2r = b2.astype(jnp.float32).reshape(1, C2)

    vec = lambda c: pl.BlockSpec((1, c), lambda p, i: (0, 0))

    out = pl.pallas_call(
        _fused_kernel,
        grid=(3, N),
        in_specs=[
            pl.BlockSpec((1, H + 2, W + 2, C1),
                         lambda p, i: (jnp.where(p == 0, i, N - 1), 0, 0, 0)),
            pl.BlockSpec((3, 3 * C1, Cm), lambda p, i: (0, 0, 0)),
            pl.BlockSpec((3, 3 * Cm, C2), lambda p, i: (0, 0, 0)),
            vec(Cm), vec(Cm), vec(C2), vec(C2),
        ],
        out_specs=pl.BlockSpec((1, H * W, C2),
                               lambda p, i: (jnp.where(p == 2, i, 0), 0, 0)),
        out_shape=jax.ShapeDtypeStruct((N, H * W, C2), jnp.float32),
        scratch_shapes=[
            pltpu.VMEM((N, H, W, Cm), _MM),
            pltpu.VMEM((N, H * W, C2), _MM),
            pltpu.VMEM((2, Cm), jnp.float32),
            pltpu.VMEM((2, C2), jnp.float32),
            pltpu.VMEM((H + 2, W, 3 * C1), _MM),
            pltpu.VMEM((H + 2, W, 3 * Cm), _MM),
        ],
        compiler_params=pltpu.CompilerParams(
            dimension_semantics=("arbitrary", "arbitrary"),
            vmem_limit_bytes=60 * 1024 * 1024),
    )(xp, w1c, w2c, g1r, b1r, g2r, b2r)

    return jnp.transpose(out.reshape(N, H, W, C2), (0, 3, 1, 2))


def kernel(x_nchw, w1, g1, b1, w2, g2, b2):
    return _double_conv(x_nchw, w1, g1, b1, w2, g2, b2)


# speedup vs baseline: 1.1216x; 1.0166x over previous
import jax
import jax.numpy as jnp
from jax.experimental import pallas as pl
from jax.experimental.pallas import tpu as pltpu

_EPS = 1e-5
_SLOPE = 0.1
_MM = jnp.bfloat16


def _tap_matmuls(cat_ref, w_ref, H, W, C):
    acc = None
    for dy in range(3):
        slab = cat_ref[dy:dy + H].reshape(H * W, 3 * C)
        d = jnp.dot(slab, w_ref[dy], preferred_element_type=jnp.float32)
        acc = d if acc is None else acc + d
    return acc


def _scale_shift(st_ref, g_ref, b_ref, r_total):
    mean = st_ref[0:1] / r_total
    var = st_ref[1:2] / r_total - mean * mean
    scale = g_ref[...] * jax.lax.rsqrt(var + _EPS)
    shift = b_ref[...] - mean * scale
    return scale, shift


def _fused_kernel(xp_ref, w1_ref, w2_ref, g1_ref, b1_ref, g2_ref, b2_ref,
                  o_ref,
                  y1_ref, y2_ref, st1_ref, st2_ref, xc_ref, hc_ref):
    p = pl.program_id(0)
    i = pl.program_id(1)
    N = pl.num_programs(1)
    H = y1_ref.shape[1]
    W = y1_ref.shape[2]
    C1 = xp_ref.shape[3]
    Cm = y1_ref.shape[3]
    C2 = y2_ref.shape[2]
    R = float(N * H * W)

    @pl.when(p == 0)
    def _phase0():
        @pl.when(i == 0)
        def _():
            st1_ref[...] = jnp.zeros_like(st1_ref)
            st2_ref[...] = jnp.zeros_like(st2_ref)
            hc_ref[0] = jnp.zeros((W, 3 * Cm), _MM)
            hc_ref[H + 1] = jnp.zeros((W, 3 * Cm), _MM)
            xc_ref[:, 0:1, 0:C1] = jnp.zeros((H + 2, 1, C1), _MM)
            xc_ref[:, W - 1:W, 2 * C1:3 * C1] = jnp.zeros((H + 2, 1, C1), _MM)
        src = xp_ref[0]
        xc_ref[:, 1:W, 0:C1] = src[:, 0:W - 1, :]
        xc_ref[:, :, C1:2 * C1] = src
        xc_ref[:, 0:W - 1, 2 * C1:3 * C1] = src[:, 1:W, :]
        acc = _tap_matmuls(xc_ref, w1_ref, H, W, C1)
        y1_ref[pl.ds(i, 1)] = acc.reshape(1, H, W, Cm).astype(_MM)
        st1_ref[0:1] = st1_ref[0:1] + jnp.sum(acc, axis=0, keepdims=True)
        st1_ref[1:2] = st1_ref[1:2] + jnp.sum(acc * acc, axis=0,
                                              keepdims=True)

    @pl.when(p == 1)
    def _phase1():
        scale, shift = _scale_shift(st1_ref, g1_ref, b1_ref, R)
        h = (y1_ref[pl.ds(i, 1)][0] * scale.reshape(1, 1, Cm).astype(_MM)
             + shift.reshape(1, 1, Cm).astype(_MM))
        h = jnp.where(h >= 0.0, h, _MM(_SLOPE) * h)
        hp = jnp.pad(h, ((0, 0), (1, 1), (0, 0)))
        for dx in range(3):
            hc_ref[1:H + 1, :, dx * Cm:(dx + 1) * Cm] = hp[:, dx:dx + W, :]
        acc = _tap_matmuls(hc_ref, w2_ref, H, W, Cm)
        y2_ref[pl.ds(i, 1)] = acc.reshape(1, H * W, C2).astype(_MM)
        st2_ref[0:1] = st2_ref[0:1] + jnp.sum(acc, axis=0, keepdims=True)
        st2_ref[1:2] = st2_ref[1:2] + jnp.sum(acc * acc, axis=0,
                                              keepdims=True)

    @pl.when(p == 2)
    def _phase2():
        scale, shift = _scale_shift(st2_ref, g2_ref, b2_ref, R)
        h = y2_ref[pl.ds(i, 1)][0].astype(jnp.float32) * scale + shift
        o_ref[0] = jnp.where(h >= 0.0, h, _SLOPE * h)


@jax.jit
def _double_conv(x_nchw, w1, g1, b1, w2, g2, b2):
    N, C1, H, W = x_nchw.shape
    Cm = w1.shape[-1]
    C2 = w2.shape[-1]

    x = jnp.transpose(x_nchw, (0, 2, 3, 1)).astype(_MM)
    xp = jnp.pad(x, ((0, 0), (1, 1), (0, 0), (0, 0)))
    w1c = w1.reshape(3, 3 * C1, Cm).astype(_MM)
    w2c = w2.reshape(3, 3 * Cm, C2).astype(_MM)
    g1r = g1.astype(jnp.float32).reshape(1, Cm)
    b1r = b1.astype(jnp.float32).reshape(1, Cm)
    g2r = g2.astype(jnp.float32).reshape(1, C2)
    b2r = b2.astype(jnp.float32).reshape(1, C2)

    vec = lambda c: pl.BlockSpec((1, c), lambda p, i: (0, 0))

    out = pl.pallas_call(
        _fused_kernel,
        grid=(3, N),
        in_specs=[
            pl.BlockSpec((1, H + 2, W, C1),
                         lambda p, i: (jnp.where(p == 0, i, N - 1), 0, 0, 0)),
            pl.BlockSpec((3, 3 * C1, Cm), lambda p, i: (0, 0, 0)),
            pl.BlockSpec((3, 3 * Cm, C2), lambda p, i: (0, 0, 0)),
            vec(Cm), vec(Cm), vec(C2), vec(C2),
        ],
        out_specs=pl.BlockSpec((1, H * W, C2),
                               lambda p, i: (jnp.where(p == 2, i, 0), 0, 0)),
        out_shape=jax.ShapeDtypeStruct((N, H * W, C2), jnp.float32),
        scratch_shapes=[
            pltpu.VMEM((N, H, W, Cm), _MM),
            pltpu.VMEM((N, H * W, C2), _MM),
            pltpu.VMEM((2, Cm), jnp.float32),
            pltpu.VMEM((2, C2), jnp.float32),
            pltpu.VMEM((H + 2, W, 3 * C1), _MM),
            pltpu.VMEM((H + 2, W, 3 * Cm), _MM),
        ],
        compiler_params=pltpu.CompilerParams(
            dimension_semantics=("arbitrary", "arbitrary"),
            vmem_limit_bytes=60 * 1024 * 1024),
    )(xp, w1c, w2c, g1r, b1r, g2r, b2r)

    return jnp.transpose(out.reshape(N, H, W, C2), (0, 3, 1, 2))


def kernel(x_nchw, w1, g1, b1, w2, g2, b2):
    return _double_conv(x_nchw, w1, g1, b1, w2, g2, b2)
